# baseline (device time: 36043 ns/iter reference)
import jax
import jax.numpy as jnp
from jax import lax
from jax.experimental import pallas as pl
from jax.experimental.pallas import tpu as pltpu

N_DEV = 32
LOG2_N = 5
N_HEADS = 8
DH = 128
SQ = 256
D_MODEL = 1024
SCALE = 0.08838834764831843

CHUNK = SQ // N_DEV


def kernel(x, Wq, Wo, Wk, Wv):
    x2 = x.reshape(SQ, D_MODEL)

    def body(x_ref, wq_ref, wo_ref, wk_ref, wv_ref, out_ref,
             acc_ref, ag_ref, rs_recv,
             rs_send_sems, rs_recv_sems, ag_send_sems, ag_recv_sems):
        my = lax.axis_index("i")

        barrier_sem = pltpu.get_barrier_semaphore()
        for j in range(N_DEV - 1):
            d = lax.rem(my + 1 + j, N_DEV)
            pl.semaphore_signal(
                barrier_sem, inc=1,
                device_id=(d,), device_id_type=pl.DeviceIdType.MESH,
            )

        xb = x_ref[:].astype(jnp.bfloat16)
        q = jnp.dot(xb, wq_ref[:].astype(jnp.bfloat16),
                    preferred_element_type=jnp.float32).astype(jnp.bfloat16)
        k = jnp.dot(xb, wk_ref[:].astype(jnp.bfloat16),
                    preferred_element_type=jnp.float32).astype(jnp.bfloat16)
        v = jnp.dot(xb, wv_ref[:].astype(jnp.bfloat16),
                    preferred_element_type=jnp.float32).astype(jnp.bfloat16)

        outs = []
        for h in range(N_HEADS):
            qh = q[:, h * DH:(h + 1) * DH]
            kh = k[:, h * DH:(h + 1) * DH]
            vh = v[:, h * DH:(h + 1) * DH]
            s = lax.dot_general(
                qh, kh, (((1,), (1,)), ((), ())),
                preferred_element_type=jnp.float32,
            ) * SCALE
            m = jnp.max(s, axis=1, keepdims=True)
            p = jnp.exp(s - m)
            l = jnp.sum(p, axis=1, keepdims=True)
            ph = p.astype(jnp.bfloat16)
            outs.append(
                jnp.dot(ph, vh, preferred_element_type=jnp.float32) / l
            )
        attn = jnp.concatenate(outs, axis=1).astype(jnp.bfloat16)
        acc_ref[:] = jnp.dot(attn, wo_ref[:].astype(jnp.bfloat16),
                             preferred_element_type=jnp.float32
                             ).astype(jnp.bfloat16)

        pl.semaphore_wait(barrier_sem, N_DEV - 1)
        rs_sends = []
        for j in range(N_DEV - 1):
            d = lax.rem(my + 1 + j, N_DEV)
            rdma = pltpu.make_async_remote_copy(
                src_ref=acc_ref.at[pl.ds(CHUNK * d, CHUNK), :],
                dst_ref=rs_recv.at[30 - j],
                send_sem=rs_send_sems.at[j],
                recv_sem=rs_recv_sems.at[30 - j],
                device_id=(d,),
                device_id_type=pl.DeviceIdType.MESH,
            )
            rdma.start()
            rs_sends.append(rdma)

        my_off = CHUNK * my
        for s in range(N_DEV - 1):
            recv = pltpu.make_async_remote_copy(
                src_ref=rs_recv.at[s],
                dst_ref=rs_recv.at[s],
                send_sem=rs_send_sems.at[0],
                recv_sem=rs_recv_sems.at[s],
                device_id=(my,),
                device_id_type=pl.DeviceIdType.MESH,
            )
            recv.wait_recv()
        red = acc_ref[pl.ds(my_off, CHUNK), :].astype(jnp.float32) + jnp.sum(
            rs_recv[:, :, :].astype(jnp.float32), axis=0
        )

        ag_ref[pl.ds(my_off, CHUNK), :] = red.astype(jnp.bfloat16)
        ag_sends = []
        for j in range(N_DEV - 1):
            d = lax.rem(my + 1 + j, N_DEV)
            rdma = pltpu.make_async_remote_copy(
                src_ref=ag_ref.at[pl.ds(my_off, CHUNK), :],
                dst_ref=ag_ref.at[pl.ds(my_off, CHUNK), :],
                send_sem=ag_send_sems.at[j],
                recv_sem=ag_recv_sems.at[30 - j],
                device_id=(d,),
                device_id_type=pl.DeviceIdType.MESH,
            )
            rdma.start()
            ag_sends.append(rdma)

        for s in range(N_DEV - 1):
            src = lax.rem(my + 1 + s, N_DEV)
            recv = pltpu.make_async_remote_copy(
                src_ref=ag_ref.at[pl.ds(CHUNK * src, CHUNK), :],
                dst_ref=ag_ref.at[pl.ds(CHUNK * src, CHUNK), :],
                send_sem=ag_send_sems.at[0],
                recv_sem=ag_recv_sems.at[s],
                device_id=(my,),
                device_id_type=pl.DeviceIdType.MESH,
            )
            recv.wait_recv()

        for rdma in rs_sends:
            rdma.wait_send()
        for rdma in ag_sends:
            rdma.wait_send()

        out_ref[0, :, :] = ag_ref[:].astype(jnp.float32)

    out = pl.pallas_call(
        body,
        out_shape=jax.ShapeDtypeStruct((1, SQ, D_MODEL), jnp.float32),
        in_specs=[pl.BlockSpec(memory_space=pltpu.VMEM)] * 5,
        out_specs=pl.BlockSpec(memory_space=pltpu.VMEM),
        scratch_shapes=[
            pltpu.VMEM((SQ, D_MODEL), jnp.bfloat16),
            pltpu.VMEM((SQ, D_MODEL), jnp.bfloat16),
            pltpu.VMEM((N_DEV - 1, CHUNK, D_MODEL), jnp.bfloat16),
            pltpu.SemaphoreType.DMA((N_DEV - 1,)),
            pltpu.SemaphoreType.DMA((N_DEV - 1,)),
            pltpu.SemaphoreType.DMA((N_DEV - 1,)),
            pltpu.SemaphoreType.DMA((N_DEV - 1,)),
        ],
        compiler_params=pltpu.CompilerParams(collective_id=0),
    )(x2, Wq, Wo, Wk, Wv)
    return out


# device time: 35079 ns/iter; 1.0275x vs baseline; 1.0275x over previous
import jax
import jax.numpy as jnp
from jax import lax
from jax.experimental import pallas as pl
from jax.experimental.pallas import tpu as pltpu

N_DEV = 32
LOG2_N = 5
N_HEADS = 8
DH = 128
SQ = 256
D_MODEL = 1024
SCALE = 0.08838834764831843

CHUNK = SQ // N_DEV


def kernel(x, Wq, Wo, Wk, Wv):
    xb = x.reshape(SQ, D_MODEL).astype(jnp.bfloat16)
    wqb = Wq.astype(jnp.bfloat16)
    wkb = Wk.astype(jnp.bfloat16)
    wvb = Wv.astype(jnp.bfloat16)
    wob = Wo.astype(jnp.bfloat16)

    def body(x_ref, wq_ref, wo_ref, wk_ref, wv_ref, out_ref,
             acc_ref, ag_ref, rs_recv,
             rs_send_sems, rs_recv_sems, ag_send_sems, ag_recv_sems):
        my = lax.axis_index("i")

        barrier_sem = pltpu.get_barrier_semaphore()
        for j in range(N_DEV - 1):
            d = lax.rem(my + 1 + j, N_DEV)
            pl.semaphore_signal(
                barrier_sem, inc=1,
                device_id=(d,), device_id_type=pl.DeviceIdType.MESH,
            )

        k = jnp.dot(x_ref[:], wk_ref[:],
                    preferred_element_type=jnp.float32).astype(jnp.bfloat16)
        v = jnp.dot(x_ref[:], wv_ref[:],
                    preferred_element_type=jnp.float32).astype(jnp.bfloat16)

        def partial_rows(r0, nrows):
            qr = jnp.dot(x_ref[pl.ds(r0, nrows), :], wq_ref[:],
                         preferred_element_type=jnp.float32
                         ).astype(jnp.bfloat16)
            outs = []
            for h in range(N_HEADS):
                qh = qr[:, h * DH:(h + 1) * DH]
                kh = k[:, h * DH:(h + 1) * DH]
                vh = v[:, h * DH:(h + 1) * DH]
                s = lax.dot_general(
                    qh, kh, (((1,), (1,)), ((), ())),
                    preferred_element_type=jnp.float32,
                ) * SCALE
                m = jnp.max(s, axis=1, keepdims=True)
                p = jnp.exp(s - m)
                l = jnp.sum(p, axis=1, keepdims=True)
                ph = p.astype(jnp.bfloat16)
                outs.append(
                    jnp.dot(ph, vh, preferred_element_type=jnp.float32) / l
                )
            attn = jnp.concatenate(outs, axis=1).astype(jnp.bfloat16)
            acc_ref[pl.ds(r0, nrows), :] = jnp.dot(
                attn, wo_ref[:], preferred_element_type=jnp.float32
            ).astype(jnp.bfloat16)

        rs_sends = []
        for j in range(N_DEV - 1):
            d = lax.rem(my + 1 + j, N_DEV)
            rdma = pltpu.make_async_remote_copy(
                src_ref=acc_ref.at[pl.ds(CHUNK * d, CHUNK), :],
                dst_ref=rs_recv.at[30 - j],
                send_sem=rs_send_sems.at[j],
                recv_sem=rs_recv_sems.at[30 - j],
                device_id=(d,),
                device_id_type=pl.DeviceIdType.MESH,
            )
            rs_sends.append((rdma, d))

        partial_rows(0, SQ // 2)
        pl.semaphore_wait(barrier_sem, N_DEV - 1)
        for rdma, d in rs_sends:

            @pl.when(d < N_DEV // 2)
            def _():
                rdma.start()

        partial_rows(SQ // 2, SQ // 2)
        for rdma, d in rs_sends:

            @pl.when(d >= N_DEV // 2)
            def _():
                rdma.start()

        my_off = CHUNK * my
        for s in range(N_DEV - 1):
            recv = pltpu.make_async_remote_copy(
                src_ref=rs_recv.at[s],
                dst_ref=rs_recv.at[s],
                send_sem=rs_send_sems.at[0],
                recv_sem=rs_recv_sems.at[s],
                device_id=(my,),
                device_id_type=pl.DeviceIdType.MESH,
            )
            recv.wait_recv()
        red = acc_ref[pl.ds(my_off, CHUNK), :].astype(jnp.float32) + jnp.sum(
            rs_recv[:, :, :].astype(jnp.float32), axis=0
        )

        ag_ref[pl.ds(my_off, CHUNK), :] = red.astype(jnp.bfloat16)
        ag_sends = []
        for j in range(N_DEV - 1):
            d = lax.rem(my + 1 + j, N_DEV)
            rdma = pltpu.make_async_remote_copy(
                src_ref=ag_ref.at[pl.ds(my_off, CHUNK), :],
                dst_ref=ag_ref.at[pl.ds(my_off, CHUNK), :],
                send_sem=ag_send_sems.at[j],
                recv_sem=ag_recv_sems.at[30 - j],
                device_id=(d,),
                device_id_type=pl.DeviceIdType.MESH,
            )
            rdma.start()
            ag_sends.append(rdma)

        for s in range(N_DEV - 1):
            src = lax.rem(my + 1 + s, N_DEV)
            recv = pltpu.make_async_remote_copy(
                src_ref=ag_ref.at[pl.ds(CHUNK * src, CHUNK), :],
                dst_ref=ag_ref.at[pl.ds(CHUNK * src, CHUNK), :],
                send_sem=ag_send_sems.at[0],
                recv_sem=ag_recv_sems.at[s],
                device_id=(my,),
                device_id_type=pl.DeviceIdType.MESH,
            )
            recv.wait_recv()

        for rdma, _d in rs_sends:
            rdma.wait_send()
        for rdma in ag_sends:
            rdma.wait_send()

        out_ref[0, :, :] = ag_ref[:].astype(jnp.float32)

    out = pl.pallas_call(
        body,
        out_shape=jax.ShapeDtypeStruct((1, SQ, D_MODEL), jnp.float32),
        in_specs=[pl.BlockSpec(memory_space=pltpu.VMEM)] * 5,
        out_specs=pl.BlockSpec(memory_space=pltpu.VMEM),
        scratch_shapes=[
            pltpu.VMEM((SQ, D_MODEL), jnp.bfloat16),
            pltpu.VMEM((SQ, D_MODEL), jnp.bfloat16),
            pltpu.VMEM((N_DEV - 1, CHUNK, D_MODEL), jnp.bfloat16),
            pltpu.SemaphoreType.DMA((N_DEV - 1,)),
            pltpu.SemaphoreType.DMA((N_DEV - 1,)),
            pltpu.SemaphoreType.DMA((N_DEV - 1,)),
            pltpu.SemaphoreType.DMA((N_DEV - 1,)),
        ],
        compiler_params=pltpu.CompilerParams(collective_id=0),
    )(xb, wqb, wob, wkb, wvb)
    return out


# device time: 34932 ns/iter; 1.0318x vs baseline; 1.0042x over previous
import jax
import jax.numpy as jnp
from jax import lax
from jax.experimental import pallas as pl
from jax.experimental.pallas import tpu as pltpu

N_DEV = 32
LOG2_N = 5
N_HEADS = 8
DH = 128
SQ = 256
D_MODEL = 1024
SCALE = 0.08838834764831843

CHUNK = SQ // N_DEV


def kernel(x, Wq, Wo, Wk, Wv):
    xb = x.reshape(SQ, D_MODEL).astype(jnp.bfloat16)
    wqb = Wq.astype(jnp.bfloat16)
    wkb = Wk.astype(jnp.bfloat16)
    wvb = Wv.astype(jnp.bfloat16)
    wob = Wo.astype(jnp.bfloat16)

    def body(x_ref, wq_ref, wo_ref, wk_ref, wv_ref, out_ref,
             acc_ref, ag_ref, rs_recv,
             rs_send_sems, rs_recv_sems, ag_send_sems, ag_recv_sems):
        my = lax.axis_index("i")

        barrier_sem = pltpu.get_barrier_semaphore()
        for j in range(N_DEV - 1):
            d = lax.rem(my + 1 + j, N_DEV)
            pl.semaphore_signal(
                barrier_sem, inc=1,
                device_id=(d,), device_id_type=pl.DeviceIdType.MESH,
            )

        q = jnp.dot(x_ref[:], wq_ref[:],
                    preferred_element_type=jnp.float32).astype(jnp.bfloat16)
        k = jnp.dot(x_ref[:], wk_ref[:],
                    preferred_element_type=jnp.float32).astype(jnp.bfloat16)
        v = jnp.dot(x_ref[:], wv_ref[:],
                    preferred_element_type=jnp.float32).astype(jnp.bfloat16)

        outs = []
        for h in range(N_HEADS):
            qh = q[:, h * DH:(h + 1) * DH]
            kh = k[:, h * DH:(h + 1) * DH]
            vh = v[:, h * DH:(h + 1) * DH]
            s = lax.dot_general(
                qh, kh, (((1,), (1,)), ((), ())),
                preferred_element_type=jnp.float32,
            ) * SCALE
            m = jnp.max(s, axis=1, keepdims=True)
            p = jnp.exp(s - m)
            l = jnp.sum(p, axis=1, keepdims=True)
            ph = p.astype(jnp.bfloat16)
            outs.append(
                jnp.dot(ph, vh, preferred_element_type=jnp.float32) / l
            )
        attn = jnp.concatenate(outs, axis=1).astype(jnp.bfloat16)
        acc_ref[:] = jnp.dot(attn, wo_ref[:],
                             preferred_element_type=jnp.float32
                             ).astype(jnp.bfloat16)

        pl.semaphore_wait(barrier_sem, N_DEV - 1)
        rs_sends = []
        for j in range(N_DEV - 1):
            d = lax.rem(my + 1 + j, N_DEV)
            rdma = pltpu.make_async_remote_copy(
                src_ref=acc_ref.at[pl.ds(CHUNK * d, CHUNK), :],
                dst_ref=rs_recv.at[30 - j],
                send_sem=rs_send_sems.at[j],
                recv_sem=rs_recv_sems.at[30 - j],
                device_id=(d,),
                device_id_type=pl.DeviceIdType.MESH,
            )
            rdma.start()
            rs_sends.append(rdma)

        my_off = CHUNK * my
        for s in range(N_DEV - 1):
            recv = pltpu.make_async_remote_copy(
                src_ref=rs_recv.at[s],
                dst_ref=rs_recv.at[s],
                send_sem=rs_send_sems.at[0],
                recv_sem=rs_recv_sems.at[s],
                device_id=(my,),
                device_id_type=pl.DeviceIdType.MESH,
            )
            recv.wait_recv()
        red = acc_ref[pl.ds(my_off, CHUNK), :].astype(jnp.float32) + jnp.sum(
            rs_recv[:, :, :].astype(jnp.float32), axis=0
        )

        ag_ref[pl.ds(my_off, CHUNK), :] = red.astype(jnp.bfloat16)
        ag_sends = []
        for j in range(N_DEV - 1):
            d = lax.rem(my + 1 + j, N_DEV)
            rdma = pltpu.make_async_remote_copy(
                src_ref=ag_ref.at[pl.ds(my_off, CHUNK), :],
                dst_ref=ag_ref.at[pl.ds(my_off, CHUNK), :],
                send_sem=ag_send_sems.at[j],
                recv_sem=ag_recv_sems.at[30 - j],
                device_id=(d,),
                device_id_type=pl.DeviceIdType.MESH,
            )
            rdma.start()
            ag_sends.append(rdma)

        for s in range(N_DEV - 1):
            src = lax.rem(my + 1 + s, N_DEV)
            recv = pltpu.make_async_remote_copy(
                src_ref=ag_ref.at[pl.ds(CHUNK * src, CHUNK), :],
                dst_ref=ag_ref.at[pl.ds(CHUNK * src, CHUNK), :],
                send_sem=ag_send_sems.at[0],
                recv_sem=ag_recv_sems.at[s],
                device_id=(my,),
                device_id_type=pl.DeviceIdType.MESH,
            )
            recv.wait_recv()

        for rdma in rs_sends:
            rdma.wait_send()
        for rdma in ag_sends:
            rdma.wait_send()

        out_ref[0, :, :] = ag_ref[:].astype(jnp.float32)

    out = pl.pallas_call(
        body,
        out_shape=jax.ShapeDtypeStruct((1, SQ, D_MODEL), jnp.float32),
        in_specs=[pl.BlockSpec(memory_space=pltpu.VMEM)] * 5,
        out_specs=pl.BlockSpec(memory_space=pltpu.VMEM),
        scratch_shapes=[
            pltpu.VMEM((SQ, D_MODEL), jnp.bfloat16),
            pltpu.VMEM((SQ, D_MODEL), jnp.bfloat16),
            pltpu.VMEM((N_DEV - 1, CHUNK, D_MODEL), jnp.bfloat16),
            pltpu.SemaphoreType.DMA((N_DEV - 1,)),
            pltpu.SemaphoreType.DMA((N_DEV - 1,)),
            pltpu.SemaphoreType.DMA((N_DEV - 1,)),
            pltpu.SemaphoreType.DMA((N_DEV - 1,)),
        ],
        compiler_params=pltpu.CompilerParams(collective_id=0),
    )(xb, wqb, wob, wkb, wvb)
    return out
